# baseline (speedup 1.0000x reference)
import dataclasses
import functools
import math

import jax
import jax.numpy as jnp
import numpy as np
from jax import lax
from jax.experimental import pallas as pl
from jax.experimental.pallas import tpu as pltpu
from jax.experimental.pallas import tpu_sc as plsc

VOCAB = 256
EMB = 128
HID = 768
B = 16
L = 2048

NC = 2
NS = 16
NW = NC * NS
NSPAN = NW // 2
SPAN_L = L // NSPAN
HHID = HID // 2
HW = HHID // 32
GROUP = 16
RING = 4
HIMASK = -65536


def _proj_body(t_ref, w_ref, b_ref, o_ref):
    o_ref[...] = (
        jnp.dot(t_ref[...], w_ref[...], preferred_element_type=jnp.float32)
        + b_ref[...]
    )


def _project(table, W, b):
    return pl.pallas_call(
        _proj_body,
        out_shape=jax.ShapeDtypeStruct((VOCAB, HID), jnp.float32),
    )(table, W, b.reshape(1, HID))


def _pack_halves_jnp(x):
    r = x.shape[0]
    u = lax.bitcast_convert_type(x.astype(jnp.bfloat16), jnp.uint16)
    u = u.reshape(r, HID // 32, 2, 16).astype(jnp.uint32)
    words = (u[:, :, 1, :] << 16) | u[:, :, 0, :]
    words = lax.bitcast_convert_type(words.reshape(r, HID // 2), jnp.int32)
    return jnp.stack([words[:, : HID // 4], words[:, HID // 4 :]])


def _pe_packed():
    position = jnp.arange(0, L, dtype=jnp.float32)[:, None]
    div_term = jnp.exp(
        jnp.arange(0, HID, 2, dtype=jnp.float32) * (-math.log(10000.0) / HID)
    )
    pe = jnp.zeros((L, HID), dtype=jnp.float32)
    pe = pe.at[:, 0::2].set(jnp.sin(position * div_term))
    pe = pe.at[:, 1::2].set(jnp.cos(position * div_term))
    return _pack_halves_jnp(pe)

_sc_mesh = plsc.VectorSubcoreMesh(core_axis_name="c", subcore_axis_name="s")

_sc_params = pltpu.CompilerParams()
if "needs_layout_passes" in pltpu.CompilerParams.__dataclass_fields__:
    _sc_params = dataclasses.replace(_sc_params, needs_layout_passes=False)


@functools.partial(
    pl.kernel,
    mesh=_sc_mesh,
    compiler_params=_sc_params,
    out_type=jax.ShapeDtypeStruct((B, L, HID), jnp.float32),
    scratch_types=[
        pltpu.VMEM((VOCAB, HID // 4), jnp.int32),
        pltpu.VMEM((SPAN_L, HID // 4), jnp.int32),
        pltpu.VMEM((SPAN_L * B,), jnp.int32),
        pltpu.VMEM((RING, GROUP, HHID), jnp.float32),
        pltpu.SemaphoreType.DMA((RING,)),
    ],
)
def _lookup(p_hbm, pe_hbm, idx_hbm, out_hbm, p_loc, pe_loc, idx_loc, buf, ssem):
    wid = lax.axis_index("s") * NC + lax.axis_index("c")
    sp = wid // 2
    hh = wid % 2

    pltpu.sync_copy(p_hbm.at[hh], p_loc)
    pltpu.sync_copy(pe_hbm.at[hh, pl.ds(sp * SPAN_L, SPAN_L)], pe_loc)
    pltpu.sync_copy(idx_hbm.at[pl.ds(sp * SPAN_L * B, SPAN_L * B)], idx_loc)

    iota = lax.iota(jnp.int32, 16)
    cols = [iota + 16 * g for g in range(HW)]

    def wait_store(k):
        pltpu.make_async_copy(
            buf.at[k],
            out_hbm.at[0, pl.ds(0, GROUP), pl.ds(0, HHID)],
            ssem.at[k],
        ).wait()

    @pl.loop(0, B)
    def _batch(bi):
        @pl.loop(0, SPAN_L // GROUP, step=RING)
        def _slot(t8):
            for k in range(RING):
                tb = t8 + k

                @pl.when(bi * (SPAN_L // GROUP) + tb >= RING)
                def _drain():
                    wait_store(k)

                @pl.loop(0, GROUP)
                def _row(j):
                    lrow = tb * GROUP + j
                    pos = bi * SPAN_L + lrow
                    vb = plsc.load_gather(
                        idx_loc, [jnp.full((16,), pos, jnp.int32)]
                    )
                    for g in range(HW):
                        pw = plsc.load_gather(p_loc, [vb, cols[g]])
                        pev = pe_loc[lrow, pl.ds(16 * g, 16)]
                        pa = plsc.bitcast(pw << 16, jnp.float32)
                        pb = plsc.bitcast(pw & HIMASK, jnp.float32)
                        ea = plsc.bitcast(pev << 16, jnp.float32)
                        eb = plsc.bitcast(pev & HIMASK, jnp.float32)
                        buf[k, j, pl.ds(32 * g, 16)] = pa + ea
                        buf[k, j, pl.ds(32 * g + 16, 16)] = pb + eb

                pltpu.async_copy(
                    buf.at[k],
                    out_hbm.at[
                        bi,
                        pl.ds(sp * SPAN_L + tb * GROUP, GROUP),
                        pl.ds(hh * HHID, HHID),
                    ],
                    ssem.at[k],
                )

    for k in range(RING):
        wait_store(k)


def kernel(phonemes, table, W, b):
    P = _project(table, W, b)
    p_pk = _pack_halves_jnp(P)
    pe_pk = _pe_packed()
    idx = (
        phonemes.astype(jnp.int32)
        .reshape(B, NSPAN, SPAN_L)
        .transpose(1, 0, 2)
        .reshape(NSPAN * B * SPAN_L)
    )
    return _lookup(p_pk, pe_pk, idx)


# speedup vs baseline: 1.1165x; 1.1165x over previous
import dataclasses
import functools
import math

import jax
import jax.numpy as jnp
from jax import lax
from jax.experimental import pallas as pl
from jax.experimental.pallas import tpu as pltpu
from jax.experimental.pallas import tpu_sc as plsc

VOCAB = 256
EMB = 128
HID = 768
B = 16
L = 2048

NC = 2
NS = 16
NW = NC * NS
LSPAN = L // NW
HW = HID // 32
PKW = HID // 2
CHUNK = 16
RING = 4
NCHUNK = B * (LSPAN // CHUNK)
DIST = 2
HIMASK = -65536


def _proj_body(t_ref, w_ref, b_ref, o_ref):
    o_ref[...] = (
        jnp.dot(t_ref[...], w_ref[...], preferred_element_type=jnp.float32)
        + b_ref[...]
    )


def _project(table, W, b):
    return pl.pallas_call(
        _proj_body,
        out_shape=jax.ShapeDtypeStruct((VOCAB, HID), jnp.float32),
    )(table, W, b.reshape(1, HID))


def _pack_words(x):
    r = x.shape[0]
    u = lax.bitcast_convert_type(x.astype(jnp.bfloat16), jnp.uint16)
    u = u.reshape(r, HID // 32, 2, 16).astype(jnp.uint32)
    words = (u[:, :, 1, :] << 16) | u[:, :, 0, :]
    return lax.bitcast_convert_type(words.reshape(r, HID // 2), jnp.int32)


def _pe_packed():
    position = jnp.arange(0, L, dtype=jnp.float32)[:, None]
    div_term = jnp.exp(
        jnp.arange(0, HID, 2, dtype=jnp.float32) * (-math.log(10000.0) / HID)
    )
    pe = jnp.zeros((L, HID), dtype=jnp.float32)
    pe = pe.at[:, 0::2].set(jnp.sin(position * div_term))
    pe = pe.at[:, 1::2].set(jnp.cos(position * div_term))
    return _pack_words(pe)


_sc_mesh = plsc.VectorSubcoreMesh(core_axis_name="c", subcore_axis_name="s")

_sc_params = pltpu.CompilerParams()
if "needs_layout_passes" in pltpu.CompilerParams.__dataclass_fields__:
    _sc_params = dataclasses.replace(_sc_params, needs_layout_passes=False)


@functools.partial(
    pl.kernel,
    mesh=_sc_mesh,
    compiler_params=_sc_params,
    out_type=jax.ShapeDtypeStruct((B, L, HID), jnp.float32),
    scratch_types=[
        pltpu.VMEM((LSPAN, PKW), jnp.int32),
        pltpu.VMEM((B * LSPAN,), jnp.int32),
        pltpu.VMEM((RING, CHUNK, PKW), jnp.int32),
        pltpu.VMEM((RING, CHUNK, HID), jnp.float32),
        pltpu.SemaphoreType.DMA((RING,)),
        pltpu.SemaphoreType.DMA((RING,)),
    ],
)
def _lookup(p_hbm, pe_hbm, idx_hbm, out_hbm, pe_loc, idx_loc, gbuf, obuf,
            gsem, ssem):
    wid = lax.axis_index("s") * NC + lax.axis_index("c")
    l0 = wid * LSPAN

    def issue_gather(c, k):
        pltpu.async_copy(
            p_hbm.at[idx_loc.at[pl.ds(c * CHUNK, CHUNK)]],
            gbuf.at[k],
            gsem.at[k],
        )

    def wait_gather(k):
        pltpu.make_async_copy(
            p_hbm.at[pl.ds(0, CHUNK)], gbuf.at[k], gsem.at[k]
        ).wait()

    def wait_store(k):
        pltpu.make_async_copy(
            obuf.at[k],
            out_hbm.at[0, pl.ds(0, CHUNK)],
            ssem.at[k],
        ).wait()

    pltpu.sync_copy(idx_hbm.at[pl.ds(wid * (B * LSPAN), B * LSPAN)], idx_loc)
    for c0 in range(DIST):
        issue_gather(c0, c0)
    pltpu.sync_copy(pe_hbm.at[pl.ds(l0, LSPAN)], pe_loc)

    @pl.loop(0, NCHUNK, step=RING)
    def _slot(t):
        bi = t // RING
        for k in range(RING):
            tt = t + k
            nx = tt + DIST
            kn = (k + DIST) % RING

            @pl.when(nx < NCHUNK)
            def _ahead():
                issue_gather(nx, kn)

            @pl.when(tt >= RING)
            def _drain():
                wait_store(k)

            wait_gather(k)

            @pl.loop(0, CHUNK)
            def _row(j):
                lrow = k * CHUNK + j
                for g in range(HW):
                    pw = gbuf[k, j, pl.ds(16 * g, 16)]
                    pev = pe_loc[lrow, pl.ds(16 * g, 16)]
                    pa = plsc.bitcast(pw << 16, jnp.float32)
                    pb = plsc.bitcast(pw & HIMASK, jnp.float32)
                    ea = plsc.bitcast(pev << 16, jnp.float32)
                    eb = plsc.bitcast(pev & HIMASK, jnp.float32)
                    obuf[k, j, pl.ds(32 * g, 16)] = pa + ea
                    obuf[k, j, pl.ds(32 * g + 16, 16)] = pb + eb

            pltpu.async_copy(
                obuf.at[k],
                out_hbm.at[bi, pl.ds(l0 + k * CHUNK, CHUNK)],
                ssem.at[k],
            )

    for k in range(RING):
        wait_store(k)


def kernel(phonemes, table, W, b):
    P = _project(table, W, b)
    p_pk = _pack_words(P)
    pe_pk = _pe_packed()
    idx = (
        phonemes.astype(jnp.int32)
        .reshape(B, NW, LSPAN)
        .transpose(1, 0, 2)
        .reshape(NW * B * LSPAN)
    )
    return _lookup(p_pk, pe_pk, idx)
